# baseline (speedup 1.0000x reference)
import dataclasses
import functools

import jax
import jax.numpy as jnp
from jax.experimental import pallas as pl
from jax.experimental.pallas import tpu as pltpu
from jax.experimental.pallas import tpu_sc as plsc

B = 4096
N_DENSE = 13
N_SPARSE = 26
VOCAB = 100000
EMB = 16

BT = 512
_DIAG_XLA_GATHER = False
SLAB = 128


def _sc_gather(table128, slab_idx, off_idx, n_rows):
    n_workers = 32
    b_per_w = n_rows // n_workers
    chunk = 416
    n_chunks = b_per_w // chunk

    cp = pltpu.CompilerParams()
    if "needs_layout_passes" in pltpu.CompilerParams.__dataclass_fields__:
        cp = dataclasses.replace(cp, needs_layout_passes=False)

    @functools.partial(
        pl.kernel,
        out_type=jax.ShapeDtypeStruct((n_rows, EMB), table128.dtype),
        mesh=plsc.VectorSubcoreMesh(core_axis_name="c", subcore_axis_name="s"),
        compiler_params=cp,
        scratch_types=[
            pltpu.VMEM((chunk,), jnp.int32),
            pltpu.VMEM((chunk,), jnp.int32),
            pltpu.VMEM((chunk, SLAB), jnp.float32),
            pltpu.VMEM((chunk, EMB), jnp.float32),
            pltpu.SemaphoreType.DMA,
        ],
    )
    def sc_kernel(table_hbm, slab_hbm, off_hbm, out_hbm,
                  idx_v, off_v, rows_v, out_v, sem):
        wid = jax.lax.axis_index("s") * 2 + jax.lax.axis_index("c")
        base = wid * b_per_w

        @pl.loop(0, n_chunks)
        def _(c):
            cb = base + c * chunk
            pltpu.sync_copy(slab_hbm.at[pl.ds(cb, chunk)], idx_v)
            pltpu.sync_copy(off_hbm.at[pl.ds(cb, chunk)], off_v)
            pltpu.async_copy(table_hbm.at[idx_v], rows_v, sem).wait()

            @pl.loop(0, chunk, step=16)
            def _(r0):
                r_vec = jax.lax.iota(jnp.int32, 16) + r0
                col0 = off_v[pl.ds(r0, 16)] * EMB
                for ed in range(EMB):
                    vals = plsc.load_gather(rows_v, [r_vec, col0 + ed])
                    plsc.store_scatter(out_v, [r_vec, r_vec * 0 + ed], vals)

            pltpu.sync_copy(out_v, out_hbm.at[pl.ds(cb, chunk)])

    return sc_kernel(table128, slab_idx, off_idx)


def _tc_body(e_ref, dense_ref, wc1_ref, cb1_ref, wc2_ref, cb2_ref,
             cf1_ref, cf2_ref, w1e_ref, w1d_ref, b1_ref, w2_ref, b2_ref,
             w3_ref, b3_ref, fw_ref, lw_ref, ball_ref, out_ref):
    f32 = jnp.float32
    dot = functools.partial(jnp.dot, preferred_element_type=f32)
    e2 = e_ref[...]
    dense = dense_ref[...]

    eT = e2.T
    denseT = dense.T
    eT3 = eT.reshape(N_SPARSE, EMB, BT)

    wc1 = wc1_ref[...]
    wc2 = wc2_ref[...]
    cb1 = cb1_ref[...]
    cb2 = cb2_ref[...]
    cin1 = jnp.zeros((N_SPARSE, BT), dtype=f32)
    cin2 = jnp.zeros((N_SPARSE, BT), dtype=f32)
    for ed in range(EMB):
        x_ed = eT3[:, ed, :]
        g1 = dot(wc1, x_ed)
        acc1 = jnp.zeros((N_SPARSE, BT), dtype=f32)
        for i in range(N_SPARSE):
            acc1 = acc1 + g1[i * N_SPARSE:(i + 1) * N_SPARSE, :] * x_ed[i:i + 1, :]
        t1 = jnp.maximum(acc1 + cb1, 0.0)
        g2 = dot(wc2, t1)
        acc2 = jnp.zeros((N_SPARSE, BT), dtype=f32)
        for i in range(N_SPARSE):
            acc2 = acc2 + g2[i * N_SPARSE:(i + 1) * N_SPARSE, :] * x_ed[i:i + 1, :]
        t2 = jnp.maximum(acc2 + cb2, 0.0)
        cin1 = cin1 + t1
        cin2 = cin2 + t2
    cin_out = dot(cf1_ref[...], cin1) + dot(cf2_ref[...], cin2)

    h2 = dot(w1e_ref[...], eT) + dot(w1d_ref[...], denseT)
    h2 = jnp.maximum(h2 + b1_ref[...], 0.0)
    h2 = jnp.maximum(dot(w2_ref[...], h2) + b2_ref[...], 0.0)
    h2 = jnp.maximum(dot(w3_ref[...], h2) + b3_ref[...], 0.0)
    deep = dot(fw_ref[...], h2)
    lin = dot(lw_ref[...], denseT)
    out_ref[...] = jax.nn.sigmoid(lin + cin_out + deep + ball_ref[...])


def _tc_forward(e2d, dense, wc1, cb1, wc2, cb2, cf1, cf2, w1e, w1d,
                b1, w2, b2, w3, b3, fw, lw, ball):
    grid = (B // BT,)
    full = lambda a: pl.BlockSpec(a.shape, lambda i: (0,) * a.ndim)
    in_specs = [
        pl.BlockSpec((BT, N_SPARSE * EMB), lambda i: (i, 0)),
        pl.BlockSpec((BT, N_DENSE), lambda i: (i, 0)),
        full(wc1), full(cb1), full(wc2), full(cb2), full(cf1), full(cf2),
        full(w1e), full(w1d), full(b1), full(w2), full(b2), full(w3),
        full(b3), full(fw), full(lw), full(ball),
    ]
    out_spec = pl.BlockSpec((1, BT), lambda i: (0, i))
    return pl.pallas_call(
        _tc_body,
        grid=grid,
        in_specs=in_specs,
        out_specs=out_spec,
        out_shape=jax.ShapeDtypeStruct((1, B), jnp.float32),
    )(e2d, dense, wc1, cb1, wc2, cb2, cf1, cf2, w1e, w1d, b1, w2,
      b2, w3, b3, fw, lw, ball)


def kernel(x, embed_tables, lin_W, lin_b, cin_W1, cin_b1, cin_W2, cin_b2,
           cin_fc_W, cin_fc_b, dnn_W1, dnn_b1, dnn_W2, dnn_b2, dnn_W3,
           dnn_b3, fin_W, fin_b):
    dense = x[:, :N_DENSE]
    sparse_idx = x[:, N_DENSE:].astype(jnp.int32)
    slab_idx = (sparse_idx // 8
                + jnp.arange(N_SPARSE, dtype=jnp.int32)[None, :] * (VOCAB // 8))
    off_idx = sparse_idx % 8
    table128 = embed_tables.reshape(N_SPARSE * VOCAB * EMB // SLAB, SLAB) * 1.0

    e_rows = _sc_gather(table128, slab_idx.reshape(-1),
                        off_idx.reshape(-1), B * N_SPARSE)
    if _DIAG_XLA_GATHER:
        e_rows = jnp.zeros((B * N_SPARSE, EMB), jnp.float32)
    e2d = e_rows.reshape(B, N_SPARSE * EMB)

    wc1 = cin_W1.reshape(N_SPARSE, N_SPARSE, N_SPARSE).transpose(1, 0, 2) \
                .reshape(N_SPARSE * N_SPARSE, N_SPARSE)
    wc2 = cin_W2.reshape(N_SPARSE, N_SPARSE, N_SPARSE).transpose(1, 0, 2) \
                .reshape(N_SPARSE * N_SPARSE, N_SPARSE)
    cb1 = cin_b1.reshape(N_SPARSE, 1)
    cb2 = cin_b2.reshape(N_SPARSE, 1)
    cf1 = cin_fc_W[:N_SPARSE].reshape(1, N_SPARSE)
    cf2 = cin_fc_W[N_SPARSE:].reshape(1, N_SPARSE)
    w1e = dnn_W1[:N_SPARSE * EMB].T
    w1d = dnn_W1[N_SPARSE * EMB:].T
    b1 = dnn_b1.reshape(-1, 1)
    b2 = dnn_b2.reshape(-1, 1)
    b3 = dnn_b3.reshape(-1, 1)
    w2 = dnn_W2.T
    w3 = dnn_W3.T
    fw = fin_W.reshape(1, -1)
    lw = lin_W.reshape(1, -1)
    ball = (lin_b + cin_fc_b + fin_b).reshape(1, 1)

    out = _tc_forward(e2d, dense, wc1, cb1, wc2, cb2, cf1, cf2,
                      w1e, w1d, b1, w2, b2, w3, b3, fw, lw, ball)
    return out.reshape(B)


# speedup vs baseline: 1.0231x; 1.0231x over previous
import dataclasses
import functools

import jax
import jax.numpy as jnp
from jax.experimental import pallas as pl
from jax.experimental.pallas import tpu as pltpu
from jax.experimental.pallas import tpu_sc as plsc

B = 4096
N_DENSE = 13
N_SPARSE = 26
VOCAB = 100000
EMB = 16

BT = 512
_DIAG_XLA_GATHER = False
SLAB = 128


def _sc_gather(table1d, elem_idx, n_rows):
    n_workers = 32
    n_elems = n_rows * EMB
    e_per_w = n_elems // n_workers
    chunk = 13312
    n_chunks = e_per_w // chunk

    cp = pltpu.CompilerParams()
    if "needs_layout_passes" in pltpu.CompilerParams.__dataclass_fields__:
        cp = dataclasses.replace(cp, needs_layout_passes=False)

    @functools.partial(
        pl.kernel,
        out_type=jax.ShapeDtypeStruct((n_elems,), table1d.dtype),
        mesh=plsc.VectorSubcoreMesh(core_axis_name="c", subcore_axis_name="s"),
        compiler_params=cp,
        scratch_types=[
            pltpu.VMEM((chunk,), jnp.int32),
            pltpu.VMEM((chunk,), jnp.float32),
            pltpu.SemaphoreType.DMA,
        ],
    )
    def sc_kernel(table_hbm, idx_hbm, out_hbm, idx_v, vals_v, sem):
        wid = jax.lax.axis_index("s") * 2 + jax.lax.axis_index("c")
        base = wid * e_per_w

        @pl.loop(0, n_chunks)
        def _(c):
            cb = base + c * chunk
            pltpu.sync_copy(idx_hbm.at[pl.ds(cb, chunk)], idx_v)
            pltpu.async_copy(table_hbm.at[idx_v], vals_v, sem).wait()
            pltpu.sync_copy(vals_v, out_hbm.at[pl.ds(cb, chunk)])

    return sc_kernel(table1d, elem_idx)


def _tc_body(e_ref, dense_ref, wc1_ref, cb1_ref, wc2_ref, cb2_ref,
             cf1_ref, cf2_ref, w1e_ref, w1d_ref, b1_ref, w2_ref, b2_ref,
             w3_ref, b3_ref, fw_ref, lw_ref, ball_ref, out_ref):
    f32 = jnp.float32
    dot = functools.partial(jnp.dot, preferred_element_type=f32)
    e2 = e_ref[...]
    dense = dense_ref[...]

    eT = e2.T
    denseT = dense.T
    eT3 = eT.reshape(N_SPARSE, EMB, BT)

    wc1 = wc1_ref[...]
    wc2 = wc2_ref[...]
    cb1 = cb1_ref[...]
    cb2 = cb2_ref[...]
    cin1 = jnp.zeros((N_SPARSE, BT), dtype=f32)
    cin2 = jnp.zeros((N_SPARSE, BT), dtype=f32)
    for ed in range(EMB):
        x_ed = eT3[:, ed, :]
        g1 = dot(wc1, x_ed)
        acc1 = jnp.zeros((N_SPARSE, BT), dtype=f32)
        for i in range(N_SPARSE):
            acc1 = acc1 + g1[i * N_SPARSE:(i + 1) * N_SPARSE, :] * x_ed[i:i + 1, :]
        t1 = jnp.maximum(acc1 + cb1, 0.0)
        g2 = dot(wc2, t1)
        acc2 = jnp.zeros((N_SPARSE, BT), dtype=f32)
        for i in range(N_SPARSE):
            acc2 = acc2 + g2[i * N_SPARSE:(i + 1) * N_SPARSE, :] * x_ed[i:i + 1, :]
        t2 = jnp.maximum(acc2 + cb2, 0.0)
        cin1 = cin1 + t1
        cin2 = cin2 + t2
    cin_out = dot(cf1_ref[...], cin1) + dot(cf2_ref[...], cin2)

    h2 = dot(w1e_ref[...], eT) + dot(w1d_ref[...], denseT)
    h2 = jnp.maximum(h2 + b1_ref[...], 0.0)
    h2 = jnp.maximum(dot(w2_ref[...], h2) + b2_ref[...], 0.0)
    h2 = jnp.maximum(dot(w3_ref[...], h2) + b3_ref[...], 0.0)
    deep = dot(fw_ref[...], h2)
    lin = dot(lw_ref[...], denseT)
    out_ref[...] = jax.nn.sigmoid(lin + cin_out + deep + ball_ref[...])


def _tc_forward(e2d, dense, wc1, cb1, wc2, cb2, cf1, cf2, w1e, w1d,
                b1, w2, b2, w3, b3, fw, lw, ball):
    grid = (B // BT,)
    full = lambda a: pl.BlockSpec(a.shape, lambda i: (0,) * a.ndim)
    in_specs = [
        pl.BlockSpec((BT, N_SPARSE * EMB), lambda i: (i, 0)),
        pl.BlockSpec((BT, N_DENSE), lambda i: (i, 0)),
        full(wc1), full(cb1), full(wc2), full(cb2), full(cf1), full(cf2),
        full(w1e), full(w1d), full(b1), full(w2), full(b2), full(w3),
        full(b3), full(fw), full(lw), full(ball),
    ]
    out_spec = pl.BlockSpec((1, BT), lambda i: (0, i))
    return pl.pallas_call(
        _tc_body,
        grid=grid,
        in_specs=in_specs,
        out_specs=out_spec,
        out_shape=jax.ShapeDtypeStruct((1, B), jnp.float32),
    )(e2d, dense, wc1, cb1, wc2, cb2, cf1, cf2, w1e, w1d, b1, w2,
      b2, w3, b3, fw, lw, ball)


def kernel(x, embed_tables, lin_W, lin_b, cin_W1, cin_b1, cin_W2, cin_b2,
           cin_fc_W, cin_fc_b, dnn_W1, dnn_b1, dnn_W2, dnn_b2, dnn_W3,
           dnn_b3, fin_W, fin_b):
    dense = x[:, :N_DENSE]
    sparse_idx = x[:, N_DENSE:].astype(jnp.int32)
    row_idx = (sparse_idx
               + jnp.arange(N_SPARSE, dtype=jnp.int32)[None, :] * VOCAB)
    elem_idx = (row_idx[:, :, None] * EMB
                + jnp.arange(EMB, dtype=jnp.int32)[None, None, :])
    table1d = embed_tables.reshape(N_SPARSE * VOCAB * EMB)

    e_flat = _sc_gather(table1d, elem_idx.reshape(-1), B * N_SPARSE)
    if _DIAG_XLA_GATHER:
        e_flat = jnp.zeros((B * N_SPARSE * EMB,), jnp.float32)
    e2d = e_flat.reshape(B, N_SPARSE * EMB)

    wc1 = cin_W1.reshape(N_SPARSE, N_SPARSE, N_SPARSE).transpose(1, 0, 2) \
                .reshape(N_SPARSE * N_SPARSE, N_SPARSE)
    wc2 = cin_W2.reshape(N_SPARSE, N_SPARSE, N_SPARSE).transpose(1, 0, 2) \
                .reshape(N_SPARSE * N_SPARSE, N_SPARSE)
    cb1 = cin_b1.reshape(N_SPARSE, 1)
    cb2 = cin_b2.reshape(N_SPARSE, 1)
    cf1 = cin_fc_W[:N_SPARSE].reshape(1, N_SPARSE)
    cf2 = cin_fc_W[N_SPARSE:].reshape(1, N_SPARSE)
    w1e = dnn_W1[:N_SPARSE * EMB].T
    w1d = dnn_W1[N_SPARSE * EMB:].T
    b1 = dnn_b1.reshape(-1, 1)
    b2 = dnn_b2.reshape(-1, 1)
    b3 = dnn_b3.reshape(-1, 1)
    w2 = dnn_W2.T
    w3 = dnn_W3.T
    fw = fin_W.reshape(1, -1)
    lw = lin_W.reshape(1, -1)
    ball = (lin_b + cin_fc_b + fin_b).reshape(1, 1)

    out = _tc_forward(e2d, dense, wc1, cb1, wc2, cb2, cf1, cf2,
                      w1e, w1d, b1, w2, b2, w3, b3, fw, lw, ball)
    return out.reshape(B)
